# baseline (speedup 1.0000x reference)
import jax
import jax.numpy as jnp
from jax import lax
from jax.experimental import pallas as pl
from jax.experimental.pallas import tpu as pltpu
from jax.experimental.pallas import tpu_sc as plsc

N_NODES = 50000
HIDDEN = 256
HALF = 128
GRAPHS = 512
OUT = 128

NC = 2
NS = 16
WPH = NC * 8
NBLK = N_NODES // 8
RPB = 390
RPW = RPB * 8
BCH = 28
NFULL = 13
TAILB = RPB - NFULL * BCH
XTRA_B = NBLK - WPH * RPB
XTRA = XTRA_B * 8
LANES = 16
NVEC = HALF // LANES
IDXT = RPW


def _sc_body(x4_hbm, idx_hbm, out_hbm, idx_v, rows_a, rows_b, acc, sem_a, sem_b):
    c = lax.axis_index("c")
    s = lax.axis_index("s")
    h = s // 8
    q = c * 8 + lax.rem(s, 8)

    def zero_row(i, carry):
        acc[pl.ds(i * LANES, LANES)] = jnp.zeros((LANES,), jnp.float32)
        return carry

    lax.fori_loop(0, GRAPHS * HALF // LANES, zero_row, 0)

    pltpu.sync_copy(idx_hbm.at[pl.ds(q * RPW, RPW)], idx_v.at[pl.ds(0, RPW)])
    pltpu.sync_copy(
        idx_hbm.at[pl.ds(WPH * RPW, XTRA)], idx_v.at[pl.ds(IDXT, XTRA)]
    )

    bufs = (rows_a, rows_b)
    sems = (sem_a, sem_b)
    bbase = q * RPB

    def fetch(j):
        nb = BCH if j < NFULL else TAILB
        src = x4_hbm.at[pl.ds(bbase + j * BCH, nb), h]
        dst = bufs[j % 2] if nb == BCH else bufs[j % 2].at[pl.ds(0, nb)]
        return pltpu.async_copy(src, dst, sems[j % 2])

    zero8 = tuple(jnp.zeros((LANES,), jnp.float32) for _ in range(NVEC))
    carry = (idx_v[pl.ds(0, LANES)][0],) + zero8

    pending = fetch(0)
    for j in range(NFULL + 1):
        nb = BCH if j < NFULL else TAILB
        start = j * BCH * 8
        rows_v = bufs[j % 2]
        nxt = fetch(j + 1) if j < NFULL else None
        pending.wait()
        pending = nxt

        def row_body(r, carry):
            prev = carry[0]
            a = carry[1:]
            seg = idx_v[pl.ds(start + r, LANES)][0]
            flush = seg != prev

            @pl.when(flush)
            def _():
                for k in range(NVEC):
                    acc[pl.ds(prev * HALF + k * LANES, LANES)] = a[k]

            fv = jnp.full((LANES,), flush)
            new_a = tuple(
                jnp.where(
                    fv,
                    rows_v[r // 8, lax.rem(r, 8), pl.ds(k * LANES, LANES)],
                    a[k] + rows_v[r // 8, lax.rem(r, 8), pl.ds(k * LANES, LANES)],
                )
                for k in range(NVEC)
            )
            return (seg,) + new_a

        def group_body(g, carry):
            prev = carry[0]
            seg_vec = idx_v[pl.ds(start + g * LANES, LANES)]
            n_same = plsc.all_reduce_population_count(
                seg_vec == jnp.full((LANES,), prev, jnp.int32)
            )[0]

            def fast(carry):
                a = carry[1:]
                new_a = []
                for k in range(NVEC):
                    d = [
                        rows_v[2 * g + i // 8, i % 8, pl.ds(k * LANES, LANES)]
                        for i in range(LANES)
                    ]
                    while len(d) > 1:
                        d = [d[i] + d[i + 1] for i in range(0, len(d), 2)]
                    new_a.append(a[k] + d[0])
                return (carry[0],) + tuple(new_a)

            def slow(carry):
                return lax.fori_loop(
                    0, LANES, lambda i, cc: row_body(g * LANES + i, cc), carry
                )

            return lax.cond(n_same == LANES, fast, slow, carry)

        carry = lax.fori_loop(0, nb * 8 // LANES, group_body, carry)

    prev = carry[0]
    for k in range(NVEC):
        acc[pl.ds(prev * HALF + k * LANES, LANES)] = carry[1 + k]

    @pl.when(q == 0)
    def _():
        pltpu.sync_copy(
            x4_hbm.at[pl.ds(WPH * RPB, XTRA_B), h],
            rows_a.at[pl.ds(0, XTRA_B)],
        )

        def tail_row(r, carry):
            prev = carry[0]
            a = carry[1:]
            seg = idx_v[pl.ds(IDXT + r, LANES)][0]
            flush = seg != prev

            @pl.when(flush)
            def _():
                for k in range(NVEC):
                    sl = pl.ds(prev * HALF + k * LANES, LANES)
                    acc[sl] = acc[sl] + a[k]

            fv = jnp.full((LANES,), flush)
            new_a = tuple(
                jnp.where(
                    fv,
                    rows_a[r // 8, lax.rem(r, 8), pl.ds(k * LANES, LANES)],
                    a[k] + rows_a[r // 8, lax.rem(r, 8), pl.ds(k * LANES, LANES)],
                )
                for k in range(NVEC)
            )
            return (seg,) + new_a

        tcarry = (idx_v[pl.ds(IDXT, LANES)][0],) + zero8
        tcarry = lax.fori_loop(0, XTRA, tail_row, tcarry)
        tprev = tcarry[0]
        for k in range(NVEC):
            sl = pl.ds(tprev * HALF + k * LANES, LANES)
            acc[sl] = acc[sl] + tcarry[1 + k]

    pltpu.sync_copy(acc, out_hbm.at[c].at[s])


def _segment_sum_sc(x4, idx):
    mesh = plsc.VectorSubcoreMesh(
        core_axis_name="c", subcore_axis_name="s", num_cores=NC, num_subcores=NS
    )
    return pl.kernel(
        _sc_body,
        out_type=jax.ShapeDtypeStruct((NC, NS, GRAPHS * HALF), jnp.float32),
        mesh=mesh,
        compiler_params=pltpu.CompilerParams(
            use_tc_tiling_on_sc=False, needs_layout_passes=False
        ),
        scratch_types=[
            pltpu.VMEM((RPW + XTRA + LANES,), jnp.int32),
            pltpu.VMEM((BCH, 8, HALF), jnp.float32),
            pltpu.VMEM((BCH, 8, HALF), jnp.float32),
            pltpu.VMEM((GRAPHS * HALF,), jnp.float32),
            pltpu.SemaphoreType.DMA,
            pltpu.SemaphoreType.DMA,
        ],
    )(x4, idx)


def _tc_body(p_ref, w_ref, b_ref, o_ref):
    p = p_ref[...].reshape(NC, NS, GRAPHS, HALF)
    lo = jnp.sum(p[:, 0:8], axis=(0, 1))
    hi = jnp.sum(p[:, 8:16], axis=(0, 1))
    pooled = jnp.concatenate([lo, hi], axis=1)
    o_ref[...] = (
        lax.dot_general(
            pooled, w_ref[...], (((1,), (1,)), ((), ())),
            preferred_element_type=jnp.float32,
        )
        + b_ref[...]
    )


def _linear_tc(parts, W, b):
    return pl.pallas_call(
        _tc_body,
        out_shape=jax.ShapeDtypeStruct((GRAPHS, OUT), jnp.float32),
    )(parts, W, b.reshape(1, OUT))


def kernel(x, batch, W, b):
    idx = batch.astype(jnp.int32)
    x4 = jnp.transpose(x.reshape(NBLK, 8, 2, HALF), (0, 2, 1, 3))
    parts = _segment_sum_sc(x4, idx)
    return _linear_tc(parts, W, b)
